# Initial kernel scaffold; baseline (speedup 1.0000x reference)
import jax
import jax.numpy as jnp
from jax.experimental import pallas as pl


def kernel(x, token_table, pos_table):
    raise NotImplementedError("write your pallas kernel here")


# speedup vs baseline: 3.6318x; 3.6318x over previous
import jax
import jax.numpy as jnp
from jax import lax
from jax.experimental import pallas as pl
from jax.experimental.pallas import tpu as pltpu
from jax.experimental.pallas import tpu_sc as plsc

_VOCAB = 100000
_MAXLEN = 200
_EMBED = 64
_BATCH = 4096

_NC = 2
_NS = 16
_NW = _NC * _NS
_SEQ_PER_W = _BATCH // _NW
_HALF = _MAXLEN // 2


def _tec_body(x2, tok, pos, out, rows0, rows1, idx0, idx1, pos_v,
              gsem0, gsem1, osem0, osem1):
    rows = (rows0, rows1)
    idx = (idx0, idx1)
    gsem = (gsem0, gsem1)
    osem = (osem0, osem1)

    wid = lax.axis_index("s") * _NC + lax.axis_index("c")
    base = wid * _SEQ_PER_W

    pltpu.sync_copy(pos, pos_v)

    def start_gather(g, b):
        pltpu.sync_copy(x2.at[pl.ds(g * 2, 2)], idx[b])
        pltpu.async_copy(tok.at[idx[b].at[0]], rows[b].at[pl.ds(0, _HALF)],
                         gsem[b])
        pltpu.async_copy(tok.at[idx[b].at[1]], rows[b].at[pl.ds(_HALF, _HALF)],
                         gsem[b])

    def wait_gather(b):
        pltpu.make_async_copy(tok.at[idx[b].at[0]],
                              rows[b].at[pl.ds(0, _HALF)], gsem[b]).wait()
        pltpu.make_async_copy(tok.at[idx[b].at[1]],
                              rows[b].at[pl.ds(_HALF, _HALF)], gsem[b]).wait()

    def start_out(g, b):
        pltpu.async_copy(rows[b], out.at[pl.ds(g * _MAXLEN, _MAXLEN)], osem[b])

    def wait_out(b):
        pltpu.make_async_copy(rows[b], out.at[pl.ds(0, _MAXLEN)],
                              osem[b]).wait()

    start_gather(base, 0)

    def outer(k, carry):
        for b in range(2):
            i = 2 * k + b
            g = base + i
            nb = 1 - b

            wait_gather(b)

            @pl.when(i + 1 < _SEQ_PER_W)
            def _prefetch():
                @pl.when(i >= 1)
                def _():
                    wait_out(nb)
                start_gather(g + 1, nb)

            rb = rows[b]

            def add_row(r, c2):
                for c in range(_EMBED // 16):
                    sl = pl.ds(c * 16, 16)
                    rb[r, sl] = rb[r, sl] + pos_v[r, sl]
                return c2

            lax.fori_loop(0, _MAXLEN, add_row, 0)

            start_out(g, b)
        return carry

    lax.fori_loop(0, _SEQ_PER_W // 2, outer, 0)
    wait_out(0)
    wait_out(1)


def kernel(x, token_table, pos_table):
    x2 = x.reshape(_BATCH * _MAXLEN // _HALF, _HALF).astype(jnp.int32)
    mesh = plsc.VectorSubcoreMesh(core_axis_name="c", subcore_axis_name="s")
    run = pl.kernel(
        _tec_body,
        out_type=jax.ShapeDtypeStruct((_BATCH * _MAXLEN, _EMBED), jnp.float32),
        mesh=mesh,
        compiler_params=pltpu.CompilerParams(use_tc_tiling_on_sc=False),
        scratch_types=[
            pltpu.VMEM((_MAXLEN, _EMBED), jnp.float32),
            pltpu.VMEM((_MAXLEN, _EMBED), jnp.float32),
            pltpu.VMEM((2, _HALF), jnp.int32),
            pltpu.VMEM((2, _HALF), jnp.int32),
            pltpu.VMEM((_MAXLEN, _EMBED), jnp.float32),
            pltpu.SemaphoreType.DMA,
            pltpu.SemaphoreType.DMA,
            pltpu.SemaphoreType.DMA,
            pltpu.SemaphoreType.DMA,
        ],
    )
    out = run(x2, token_table, pos_table)
    return out.reshape(_BATCH, _MAXLEN, _EMBED)


# speedup vs baseline: 4.1538x; 1.1437x over previous
import jax
import jax.numpy as jnp
from jax import lax
from jax.experimental import pallas as pl
from jax.experimental.pallas import tpu as pltpu
from jax.experimental.pallas import tpu_sc as plsc

_VOCAB = 100000
_MAXLEN = 200
_EMBED = 64
_BATCH = 4096

_NC = 2
_NS = 16
_NW = _NC * _NS
_SEQ_PER_W = _BATCH // _NW
_CHUNK = 4
_NCHUNK = _SEQ_PER_W // _CHUNK
_SPLITS = ((0, 104), (104, 96))


def _tec_body(x, tok, pos, out, rows0, rows1, idx0, idx1, pos_v,
              gsem0, gsem1, osem0, osem1):
    rows = (rows0, rows1)
    idx = (idx0, idx1)
    gsem = (gsem0, gsem1)
    osem = (osem0, osem1)

    wid = lax.axis_index("s") * _NC + lax.axis_index("c")
    base = wid * _SEQ_PER_W

    pltpu.sync_copy(pos, pos_v)

    def start_gather(c, b):
        s0 = base + c * _CHUNK
        pltpu.sync_copy(x.at[pl.ds(s0, _CHUNK)], idx[b])
        for q in range(_CHUNK):
            for off, n in _SPLITS:
                pltpu.async_copy(
                    tok.at[idx[b].at[q, pl.ds(off, n)]],
                    rows[b].at[q, pl.ds(off, n)],
                    gsem[b])

    def wait_gather(b):
        for q in range(_CHUNK):
            for off, n in _SPLITS:
                pltpu.make_async_copy(
                    tok.at[idx[b].at[q, pl.ds(off, n)]],
                    rows[b].at[q, pl.ds(off, n)],
                    gsem[b]).wait()

    def start_out(c, b):
        s0 = base + c * _CHUNK
        pltpu.async_copy(rows[b], out.at[pl.ds(s0, _CHUNK)], osem[b])

    def wait_out(b):
        pltpu.make_async_copy(rows[b], out.at[pl.ds(0, _CHUNK)],
                              osem[b]).wait()

    start_gather(0, 0)

    def outer(k, carry):
        for b in range(2):
            c = 2 * k + b
            nb = 1 - b

            wait_gather(b)

            @pl.when(c + 1 < _NCHUNK)
            def _prefetch():
                @pl.when(c >= 1)
                def _():
                    wait_out(nb)
                start_gather(c + 1, nb)

            rb = rows[b]

            def add_pos(s, c2):
                p = [pos_v[s, pl.ds(e * 16, 16)] for e in range(_EMBED // 16)]
                for q in range(_CHUNK):
                    for e in range(_EMBED // 16):
                        sl = pl.ds(e * 16, 16)
                        rb[q, s, sl] = rb[q, s, sl] + p[e]
                return c2

            lax.fori_loop(0, _MAXLEN, add_pos, 0)

            start_out(c, b)
        return carry

    lax.fori_loop(0, _NCHUNK // 2, outer, 0)
    wait_out(0)
    wait_out(1)


def kernel(x, token_table, pos_table):
    xi = x.astype(jnp.int32)
    mesh = plsc.VectorSubcoreMesh(core_axis_name="c", subcore_axis_name="s")
    run = pl.kernel(
        _tec_body,
        out_type=jax.ShapeDtypeStruct((_BATCH, _MAXLEN, _EMBED), jnp.float32),
        mesh=mesh,
        compiler_params=pltpu.CompilerParams(use_tc_tiling_on_sc=False),
        scratch_types=[
            pltpu.VMEM((_CHUNK, _MAXLEN, _EMBED), jnp.float32),
            pltpu.VMEM((_CHUNK, _MAXLEN, _EMBED), jnp.float32),
            pltpu.VMEM((_CHUNK, _MAXLEN), jnp.int32),
            pltpu.VMEM((_CHUNK, _MAXLEN), jnp.int32),
            pltpu.VMEM((_MAXLEN, _EMBED), jnp.float32),
            pltpu.SemaphoreType.DMA,
            pltpu.SemaphoreType.DMA,
            pltpu.SemaphoreType.DMA,
            pltpu.SemaphoreType.DMA,
        ],
    )
    return run(xi, token_table, pos_table)
